# baseline (speedup 1.0000x reference)
import functools

import jax
import jax.numpy as jnp
from jax.experimental import pallas as pl
from jax.experimental.pallas import tpu as pltpu

Q = 1024
D = 128
K = 100000
KB = 2048
NBLK = (K + KB - 1) // KB
INT_MAX = jnp.iinfo(jnp.int32).max


def _nn_kernel(q_ref, k_ref, d_ref, i_ref, q2_ref, qsq_ref, lane_ref,
               *, kb, nkeys):
    i = pl.program_id(0)

    @pl.when(i == 0)
    def _init():
        d_ref[...] = jnp.full_like(d_ref, jnp.inf)
        i_ref[...] = jnp.zeros_like(i_ref)
        lane_ref[...] = jax.lax.broadcasted_iota(
            jnp.int32, (1, lane_ref.shape[1]), 1).astype(jnp.float32)
        q = q_ref[...]
        q2_ref[...] = q + q
        qsq_ref[...] = jnp.sum(q * q, axis=1, keepdims=True)

    kblk = k_ref[...]
    row = jax.lax.broadcasted_iota(jnp.int32, (kblk.shape[0], 1), 0)
    kblk = jnp.where(row + i * kb < nkeys, kblk, jnp.float32(0.0))
    qsq = qsq_ref[...]
    half = kblk.shape[0] // 2
    mins = []
    idxs = []
    for h in (0, 1):
        kh = kblk[h * half:(h + 1) * half, :]
        k_sq = jnp.sum(kh * kh, axis=1)[None, :]
        lane = jax.lax.broadcasted_iota(jnp.int32, k_sq.shape, 1)
        k_sq = jnp.where(lane + (i * kb + h * half) < nkeys, k_sq, jnp.inf)
        cross2 = jax.lax.dot_general(
            q2_ref[...], kh, (((1,), (1,)), ((), ())),
            preferred_element_type=jnp.float32,
            precision=jax.lax.Precision.DEFAULT)
        d2 = (qsq + k_sq) - cross2
        bmin_h = jnp.min(d2, axis=1, keepdims=True)
        bidx_h = jnp.min(jnp.where(d2 == bmin_h,
                                   lane_ref[:, h * half:(h + 1) * half],
                                   jnp.inf),
                         axis=1, keepdims=True)
        mins.append(bmin_h)
        idxs.append(bidx_h)
    second = mins[1] < mins[0]
    bmin = jnp.where(second, mins[1], mins[0])
    bidx_f = jnp.where(second, idxs[1], idxs[0])
    bidx = bidx_f.astype(jnp.int32) + i * kb
    s = jnp.sqrt(jnp.maximum(bmin, jnp.float32(1e-12)))

    run_d = d_ref[...]
    better = s < run_d
    d_ref[...] = jnp.where(better, s, run_d)
    i_ref[...] = jnp.where(better, bidx, i_ref[...])


def kernel(queries, keys, k):
    d_min, idx = pl.pallas_call(
        functools.partial(_nn_kernel, kb=KB, nkeys=K),
        grid=(NBLK,),
        in_specs=[
            pl.BlockSpec((Q, D), lambda i: (0, 0)),
            pl.BlockSpec((KB, D), lambda i: (i, 0)),
        ],
        out_specs=[
            pl.BlockSpec((Q, 1), lambda i: (0, 0)),
            pl.BlockSpec((Q, 1), lambda i: (0, 0)),
        ],
        out_shape=[
            jax.ShapeDtypeStruct((Q, 1), jnp.float32),
            jax.ShapeDtypeStruct((Q, 1), jnp.int32),
        ],
        scratch_shapes=[
            pltpu.VMEM((Q, D), jnp.float32),
            pltpu.VMEM((Q, 1), jnp.float32),
            pltpu.VMEM((1, KB), jnp.float32),
        ],
    )(queries, keys)
    knn_dists = d_min
    anomaly_scores = knn_dists[:, 0]
    return anomaly_scores, knn_dists, idx


# speedup vs baseline: 1.0036x; 1.0036x over previous
import functools

import jax
import jax.numpy as jnp
from jax.experimental import pallas as pl
from jax.experimental.pallas import tpu as pltpu

Q = 1024
D = 128
K = 100000
KB = 2048
NBLK = (K + KB - 1) // KB
INT_MAX = jnp.iinfo(jnp.int32).max


def _nn_kernel(q_ref, k_ref, d_ref, i_ref, q2_ref, qsq_ref, lane_ref,
               *, kb, nkeys):
    i = pl.program_id(0)

    @pl.when(i == 0)
    def _init():
        d_ref[...] = jnp.full_like(d_ref, jnp.inf)
        i_ref[...] = jnp.zeros_like(i_ref)
        lane_ref[...] = jax.lax.broadcasted_iota(
            jnp.int32, (1, lane_ref.shape[1]), 1).astype(jnp.float32)
        q = q_ref[...]
        q2_ref[...] = q + q
        qsq_ref[...] = jnp.sum(q * q, axis=1, keepdims=True)

    kblk = k_ref[...]
    row = jax.lax.broadcasted_iota(jnp.int32, (kblk.shape[0], 1), 0)
    kblk = jnp.where(row + i * kb < nkeys, kblk, jnp.float32(0.0))
    k_sq = jnp.sum(kblk * kblk, axis=1)[None, :]
    lane = jax.lax.broadcasted_iota(jnp.int32, k_sq.shape, 1)
    k_sq = jnp.where(lane + i * kb < nkeys, k_sq, jnp.inf)
    cross2 = jax.lax.dot_general(
        q2_ref[...], kblk, (((1,), (1,)), ((), ())),
        preferred_element_type=jnp.float32,
        precision=jax.lax.Precision.DEFAULT)
    d2 = (qsq_ref[...] + k_sq) - cross2

    bmin = jnp.min(d2, axis=1, keepdims=True)
    bidx_f = jnp.min(jnp.where(d2 == bmin, lane_ref[...], jnp.inf),
                     axis=1, keepdims=True)
    bidx = bidx_f.astype(jnp.int32) + i * kb
    s = jnp.sqrt(jnp.maximum(bmin, jnp.float32(1e-12)))

    run_d = d_ref[...]
    better = s < run_d
    d_ref[...] = jnp.where(better, s, run_d)
    i_ref[...] = jnp.where(better, bidx, i_ref[...])


def kernel(queries, keys, k):
    d_min, idx = pl.pallas_call(
        functools.partial(_nn_kernel, kb=KB, nkeys=K),
        grid=(NBLK,),
        in_specs=[
            pl.BlockSpec((Q, D), lambda i: (0, 0)),
            pl.BlockSpec((KB, D), lambda i: (i, 0)),
        ],
        out_specs=[
            pl.BlockSpec((Q, 1), lambda i: (0, 0)),
            pl.BlockSpec((Q, 1), lambda i: (0, 0)),
        ],
        out_shape=[
            jax.ShapeDtypeStruct((Q, 1), jnp.float32),
            jax.ShapeDtypeStruct((Q, 1), jnp.int32),
        ],
        scratch_shapes=[
            pltpu.VMEM((Q, D), jnp.float32),
            pltpu.VMEM((Q, 1), jnp.float32),
            pltpu.VMEM((1, KB), jnp.float32),
        ],
    )(queries, keys)
    knn_dists = d_min
    anomaly_scores = knn_dists[:, 0]
    return anomaly_scores, knn_dists, idx


# speedup vs baseline: 1.0041x; 1.0005x over previous
import functools

import jax
import jax.numpy as jnp
from jax.experimental import pallas as pl
from jax.experimental.pallas import tpu as pltpu

Q = 1024
D = 128
K = 100000
KB = 2048
NBLK = (K + KB - 1) // KB


def _nn_kernel(q_ref, k_ref, d_ref, i_ref, q2_ref, qsq_ref, lane_ref,
               *, kb, nkeys):
    i = pl.program_id(0)

    @pl.when(i == 0)
    def _init():
        d_ref[...] = jnp.full_like(d_ref, jnp.inf)
        i_ref[...] = jnp.zeros_like(i_ref)
        lane_ref[...] = jax.lax.broadcasted_iota(
            jnp.int32, (1, lane_ref.shape[1]), 1).astype(jnp.float32)
        q = q_ref[...]
        q2_ref[...] = q + q
        qsq_ref[...] = jnp.sum(q * q, axis=1, keepdims=True)

    kblk = k_ref[...]
    row = jax.lax.broadcasted_iota(jnp.int32, (kblk.shape[0], 1), 0)
    kblk = jnp.where(row + i * kb < nkeys, kblk, jnp.float32(0.0))
    k_sq = jnp.sum(kblk * kblk, axis=1)[None, :]
    lane = jax.lax.broadcasted_iota(jnp.int32, k_sq.shape, 1)
    k_sq = jnp.where(lane + i * kb < nkeys, k_sq, jnp.inf)
    cross2 = jax.lax.dot_general(
        q2_ref[...], kblk, (((1,), (1,)), ((), ())),
        preferred_element_type=jnp.float32,
        precision=jax.lax.Precision.DEFAULT)
    d2 = (qsq_ref[...] + k_sq) - cross2

    bmin = jnp.min(d2, axis=1, keepdims=True)
    bidx_f = jnp.min(jnp.where(d2 == bmin, lane_ref[...], jnp.inf),
                     axis=1, keepdims=True)
    bidx = bidx_f.astype(jnp.int32) + i * kb
    s = jnp.sqrt(jnp.maximum(bmin, jnp.float32(1e-12)))

    run_d = d_ref[...]
    better = s < run_d
    d_ref[...] = jnp.where(better, s, run_d)
    i_ref[...] = jnp.where(better, bidx, i_ref[...])


def kernel(queries, keys, k):
    d_min, idx = pl.pallas_call(
        functools.partial(_nn_kernel, kb=KB, nkeys=K),
        grid=(NBLK,),
        in_specs=[
            pl.BlockSpec((Q, D), lambda i: (0, 0)),
            pl.BlockSpec((KB, D), lambda i: (i, 0)),
        ],
        out_specs=[
            pl.BlockSpec((Q, 1), lambda i: (0, 0)),
            pl.BlockSpec((Q, 1), lambda i: (0, 0)),
        ],
        out_shape=[
            jax.ShapeDtypeStruct((Q, 1), jnp.float32),
            jax.ShapeDtypeStruct((Q, 1), jnp.int32),
        ],
        scratch_shapes=[
            pltpu.VMEM((Q, D), jnp.float32),
            pltpu.VMEM((Q, 1), jnp.float32),
            pltpu.VMEM((1, KB), jnp.float32),
        ],
    )(queries, keys)
    knn_dists = d_min
    anomaly_scores = knn_dists[:, 0]
    return anomaly_scores, knn_dists, idx
